# baseline (speedup 1.0000x reference)
import functools

import jax
import jax.numpy as jnp
from jax import lax
from jax.experimental import pallas as pl
from jax.experimental.pallas import tpu as pltpu
from jax.experimental.pallas import tpu_sc as plsc

_NC = 2
_NS = 16
_NW = _NC * _NS
_CH = 128
_WT = 64


def _tbl_body(f_ref, w_ref, b_ref, x_ref, o_ref):
    p = (
        jnp.dot(f_ref[...], w_ref[...], preferred_element_type=jnp.float32)
        + b_ref[...]
    )
    pb = p.astype(jnp.bfloat16).astype(jnp.float32)
    u = lax.bitcast_convert_type(pb, jnp.int32)
    word = (
        lax.shift_right_logical(u[:, 0:32], 16)
        | (u[:, 32:64] & jnp.int32(-65536))
    )
    o_ref[...] = jnp.concatenate(
        [lax.bitcast_convert_type(word, jnp.float32), x_ref[...]], axis=1)


def _make_gather(ep, cpt):
    mesh = plsc.VectorSubcoreMesh(
        core_axis_name="c", subcore_axis_name="s",
        num_cores=_NC, num_subcores=_NS,
    )

    nbuf = 2
    ns = 4
    ngroups = cpt // nbuf

    @functools.partial(
        pl.kernel,
        out_type=jax.ShapeDtypeStruct((ep, _WT), jnp.float32),
        mesh=mesh,
        scratch_types=[
            pltpu.VMEM((cpt, _CH), jnp.int32),
            pltpu.VMEM((ns, nbuf, _CH, _WT), jnp.float32),
        ] + [pltpu.SemaphoreType.DMA] * (2 * ns),
        compiler_params=pltpu.CompilerParams(use_tc_tiling_on_sc=False),
    )
    def gather_k(idx_hbm, tbl_hbm, out_hbm, idx_v, buf, *sems):
        wid = lax.axis_index("s") * _NC + lax.axis_index("c")
        row0 = wid * cpt
        pltpu.sync_copy(idx_hbm.at[pl.ds(row0, cpt)], idx_v)
        gsems = sems[:ns]
        osems = sems[ns:]

        def fire(g, s):
            for b in range(nbuf):
                i = g * nbuf + b
                pltpu.async_copy(tbl_hbm.at[idx_v.at[i]], buf.at[s, b],
                                 gsems[s])

        def wait_gathers(s):
            for b in range(nbuf):
                pltpu.make_async_copy(
                    tbl_hbm.at[pl.ds(0, _CH)], buf.at[s, b], gsems[s]).wait()

        def fire_outs(g, s):
            for b in range(nbuf):
                c = row0 + (g * nbuf + b)
                pltpu.async_copy(buf.at[s, b],
                                 out_hbm.at[pl.ds(c * _CH, _CH)], osems[s])

        def drain_outs(s):
            for b in range(nbuf):
                pltpu.make_async_copy(
                    buf.at[s, b], out_hbm.at[pl.ds(0, _CH)], osems[s]).wait()

        fire(0, 0)
        fire(1, 1)

        def body(t, carry):
            for j in range(ns):
                g = ns * t + j
                wait_gathers(j)
                fire_outs(g, j)
                s2 = (j + 2) % ns

                @pl.when((g >= 2) & (g + 2 < ngroups))
                def _():
                    drain_outs(s2)

                @pl.when(g + 2 < ngroups)
                def _():
                    fire(g + 2, s2)
            return carry

        lax.fori_loop(0, ngroups // ns, body, 0)
        for s in range(ns):
            drain_outs(s)

    return gather_k


def _make_finish(blk, k_):
    e2 = blk * k_ // 2

    def finish_body(g_ref, pc_ref, ct_ref, mpt_ref, mpr_ref, ssel_ref,
                    w1b_ref, sd_ref, b1q_ref, w2a_ref, w2b_ref, o_ref):
        g = g_ref[...]
        c2 = jnp.dot(pc_ref[...], ct_ref[...],
                     preferred_element_type=jnp.float32)
        g3 = g.reshape(blk, k_ // 2, 128)
        diffm = ((c2[:, None, :] - g3)
                 * mpt_ref[...]).reshape(e2, 128)
        ssq = jnp.dot(diffm * diffm, ssel_ref[...],
                      preferred_element_type=jnp.float32)
        dist = jnp.sqrt(ssq + 1e-12)
        g1 = (
            jnp.dot(diffm, w1b_ref[...], preferred_element_type=jnp.float32)
            + jnp.dot(dist, sd_ref[...], preferred_element_type=jnp.float32)
            + b1q_ref[...]
        )
        g1 = jnp.where(g1 >= 0, g1, 0.2 * g1)
        za = jnp.dot(g1, w2a_ref[...], preferred_element_type=jnp.float32)
        zb = jnp.dot(g1, w2b_ref[...], preferred_element_type=jnp.float32)
        gi = g_ref.bitcast(jnp.int32)[...]
        gpe = lax.bitcast_convert_type(gi << 16, jnp.float32) * mpr_ref[...]
        gpo = lax.bitcast_convert_type(
            gi & jnp.int32(-65536), jnp.float32) * mpr_ref[...]
        se = za + gpe
        se = jnp.where(se >= 0, se, 0.2 * se)
        so = zb + gpo
        so = jnp.where(so >= 0, so, 0.2 * so)
        sse = jnp.sum(se.reshape(blk, k_ // 2, 128), axis=1)
        sso = jnp.sum(so.reshape(blk, k_ // 2, 128), axis=1)
        outa = sse[:, 0:32] + sse[:, 64:96]
        outb = sso[:, 0:32] + sso[:, 64:96]
        o_ref[...] = jnp.concatenate([outa, outb], axis=1) * (1.0 / k_)

    return finish_body


def kernel(points, features, knn_idx, W1, b1, W2, b2):
    b_, n_, _ = points.shape
    k_ = knn_idx.shape[1]
    d_ = features.shape[-1]
    dh = W2.shape[1]
    e_ = n_ * k_

    pts = points.reshape(n_, 3)
    feats = features.reshape(n_, d_)

    w2_top = W2[:d_]
    w2_bot = W2[d_:]
    b1r = b1.reshape(1, dh)
    b2r = b2.reshape(1, dh)
    eye16 = jnp.eye(16, dtype=jnp.float32)
    w1_pad = jnp.zeros((16, dh), jnp.float32).at[:3].set(W1[:3])
    w1d = W1[3]

    f32z = functools.partial(jnp.zeros, dtype=jnp.float32)
    lanes = jnp.arange(128)
    mpt = ((lanes % 64) >= 32).astype(jnp.float32).reshape(1, 128)
    mpr = ((lanes % 64) < 32).astype(jnp.float32).reshape(1, 128)
    ct = f32z((16, 128))
    ssel = f32z((128, 2))
    w1b = f32z((128, 128))
    sd = f32z((2, 128))
    w2a = f32z((128, 128))
    w2b = f32z((128, 128))
    for t in range(2):
        o = 64 * t
        ct = ct.at[:, o + 32:o + 48].set(eye16)
        ssel = ssel.at[o + 32:o + 64, t].set(1.0)
        w1b = w1b.at[o + 32:o + 48, o:o + 64].set(w1_pad)
        sd = sd.at[t, o:o + 64].set(w1d)
        w2a = w2a.at[o:o + 64, o:o + 32].set(w2_bot[:, 0:32])
        w2b = w2b.at[o:o + 64, o:o + 32].set(w2_bot[:, 32:64])
    b1q = jnp.concatenate([b1r, b1r], axis=1)

    nph = 2
    n_ph = n_ // nph
    e_ph = n_ph * k_
    cpt = (e_ph + _NW * _CH - 1) // (_NW * _CH)
    cpt = ((cpt + 7) // 8) * 8
    ep = cpt * _NW * _CH
    idx_ph = jnp.pad(knn_idx.reshape(nph, e_ph), ((0, 0), (0, ep - e_ph)))

    pts_pad = f32z((n_, 32)).at[:, :3].set(pts)
    pts16 = pts_pad[:, :16]
    tbl = pl.pallas_call(
        _tbl_body,
        out_shape=jax.ShapeDtypeStruct((n_, 2 * dh // 2), jnp.float32),
    )(feats, w2_top, b2r, pts_pad)

    gather_fn = _make_gather(ep, cpt)
    galls = [
        gather_fn(idx_ph[p].reshape(ep // _CH, _CH), tbl).reshape(
            ep // 2, 128)
        for p in range(nph)
    ]

    blk = 200
    nb = n_ph // blk
    e2b = blk * k_ // 2
    full = lambda i: (0, 0)

    def finish_call(p):
        off = p * nb
        return pl.pallas_call(
            _make_finish(blk, k_),
            grid=(nb,),
            in_specs=[
                pl.BlockSpec((e2b, 128), lambda i: (i, 0)),
                pl.BlockSpec((blk, 16), lambda i, o=off: (i + o, 0)),
                pl.BlockSpec((16, 128), full),
                pl.BlockSpec((1, 128), full),
                pl.BlockSpec((1, 128), full),
                pl.BlockSpec((128, 2), full),
                pl.BlockSpec((128, 128), full),
                pl.BlockSpec((2, 128), full),
                pl.BlockSpec((1, 128), full),
                pl.BlockSpec((128, 128), full),
                pl.BlockSpec((128, 128), full),
            ],
            out_specs=pl.BlockSpec((blk, dh), lambda i: (i, 0)),
            out_shape=jax.ShapeDtypeStruct((n_ph, dh), jnp.float32),
        )

    outs = [
        finish_call(p)(galls[p], pts16, ct,
                       mpt, mpr, ssel, w1b, sd, b1q, w2a, w2b)
        for p in range(nph)
    ]

    return jnp.concatenate(outs, axis=0).reshape(b_, n_, dh)


# speedup vs baseline: 1.0042x; 1.0042x over previous
import functools

import jax
import jax.numpy as jnp
from jax import lax
from jax.experimental import pallas as pl
from jax.experimental.pallas import tpu as pltpu
from jax.experimental.pallas import tpu_sc as plsc

_NC = 2
_NS = 16
_NW = _NC * _NS
_CH = 128
_WT = 64


def _tbl_body(f_ref, w_ref, b_ref, x_ref, o_ref):
    p = (
        jnp.dot(f_ref[...], w_ref[...], preferred_element_type=jnp.float32)
        + b_ref[...]
    )
    pb = p.astype(jnp.bfloat16).astype(jnp.float32)
    u = lax.bitcast_convert_type(pb, jnp.int32)
    word = (
        lax.shift_right_logical(u[:, 0:32], 16)
        | (u[:, 32:64] & jnp.int32(-65536))
    )
    o_ref[...] = jnp.concatenate(
        [lax.bitcast_convert_type(word, jnp.float32), x_ref[...]], axis=1)


def _make_gather(ep, cpt):
    mesh = plsc.VectorSubcoreMesh(
        core_axis_name="c", subcore_axis_name="s",
        num_cores=_NC, num_subcores=_NS,
    )

    nbuf = 2
    ns = 4
    ngroups = cpt // nbuf

    @functools.partial(
        pl.kernel,
        out_type=jax.ShapeDtypeStruct((ep, _WT), jnp.float32),
        mesh=mesh,
        scratch_types=[
            pltpu.VMEM((cpt, _CH), jnp.int32),
            pltpu.VMEM((ns, nbuf, _CH, _WT), jnp.float32),
        ] + [pltpu.SemaphoreType.DMA] * (2 * ns),
        compiler_params=pltpu.CompilerParams(use_tc_tiling_on_sc=False),
    )
    def gather_k(idx_hbm, tbl_hbm, out_hbm, idx_v, buf, *sems):
        wid = lax.axis_index("s") * _NC + lax.axis_index("c")
        row0 = wid * cpt
        pltpu.sync_copy(idx_hbm.at[wid], idx_v)
        gsems = sems[:ns]
        osems = sems[ns:]

        def fire(g, s):
            for b in range(nbuf):
                i = g * nbuf + b
                pltpu.async_copy(tbl_hbm.at[idx_v.at[i]], buf.at[s, b],
                                 gsems[s])

        def wait_gathers(s):
            for b in range(nbuf):
                pltpu.make_async_copy(
                    tbl_hbm.at[pl.ds(0, _CH)], buf.at[s, b], gsems[s]).wait()

        def fire_outs(g, s):
            for b in range(nbuf):
                c = row0 + (g * nbuf + b)
                pltpu.async_copy(buf.at[s, b],
                                 out_hbm.at[pl.ds(c * _CH, _CH)], osems[s])

        def drain_outs(s):
            for b in range(nbuf):
                pltpu.make_async_copy(
                    buf.at[s, b], out_hbm.at[pl.ds(0, _CH)], osems[s]).wait()

        fire(0, 0)
        fire(1, 1)

        def body(t, carry):
            for j in range(ns):
                g = ns * t + j
                wait_gathers(j)
                fire_outs(g, j)
                s2 = (j + 2) % ns

                @pl.when((g >= 2) & (g + 2 < ngroups))
                def _():
                    drain_outs(s2)

                @pl.when(g + 2 < ngroups)
                def _():
                    fire(g + 2, s2)
            return carry

        lax.fori_loop(0, ngroups // ns, body, 0)
        for s in range(ns):
            drain_outs(s)

    return gather_k


def _make_finish(blk, k_):
    e2 = blk * k_ // 2

    def finish_body(g_ref, pc_ref, ct_ref, mpt_ref, mpr_ref, ssel_ref,
                    w1b_ref, sd_ref, b1q_ref, w2a_ref, w2b_ref, o_ref):
        g = g_ref[...]
        c2 = jnp.dot(pc_ref[...], ct_ref[...],
                     preferred_element_type=jnp.float32)
        g3 = g.reshape(blk, k_ // 2, 128)
        diffm = ((c2[:, None, :] - g3)
                 * mpt_ref[...]).reshape(e2, 128)
        ssq = jnp.dot(diffm * diffm, ssel_ref[...],
                      preferred_element_type=jnp.float32)
        dist = jnp.sqrt(ssq + 1e-12)
        g1 = (
            jnp.dot(diffm, w1b_ref[...], preferred_element_type=jnp.float32)
            + jnp.dot(dist, sd_ref[...], preferred_element_type=jnp.float32)
            + b1q_ref[...]
        )
        g1 = jnp.where(g1 >= 0, g1, 0.2 * g1)
        za = jnp.dot(g1, w2a_ref[...], preferred_element_type=jnp.float32)
        zb = jnp.dot(g1, w2b_ref[...], preferred_element_type=jnp.float32)
        gi = g_ref.bitcast(jnp.int32)[...]
        gpe = lax.bitcast_convert_type(gi << 16, jnp.float32) * mpr_ref[...]
        gpo = lax.bitcast_convert_type(
            gi & jnp.int32(-65536), jnp.float32) * mpr_ref[...]
        se = za + gpe
        se = jnp.where(se >= 0, se, 0.2 * se)
        so = zb + gpo
        so = jnp.where(so >= 0, so, 0.2 * so)
        sse = jnp.sum(se.reshape(blk, k_ // 2, 128), axis=1)
        sso = jnp.sum(so.reshape(blk, k_ // 2, 128), axis=1)
        outa = sse[:, 0:32] + sse[:, 64:96]
        outb = sso[:, 0:32] + sso[:, 64:96]
        o_ref[...] = jnp.concatenate([outa, outb], axis=1) * (1.0 / k_)

    return finish_body


def kernel(points, features, knn_idx, W1, b1, W2, b2):
    b_, n_, _ = points.shape
    k_ = knn_idx.shape[1]
    d_ = features.shape[-1]
    dh = W2.shape[1]
    e_ = n_ * k_

    pts = points.reshape(n_, 3)
    feats = features.reshape(n_, d_)

    w2_top = W2[:d_]
    w2_bot = W2[d_:]
    b1r = b1.reshape(1, dh)
    b2r = b2.reshape(1, dh)
    eye16 = jnp.eye(16, dtype=jnp.float32)
    w1_pad = jnp.zeros((16, dh), jnp.float32).at[:3].set(W1[:3])
    w1d = W1[3]

    f32z = functools.partial(jnp.zeros, dtype=jnp.float32)
    lanes = jnp.arange(128)
    mpt = ((lanes % 64) >= 32).astype(jnp.float32).reshape(1, 128)
    mpr = ((lanes % 64) < 32).astype(jnp.float32).reshape(1, 128)
    ct = f32z((16, 128))
    ssel = f32z((128, 2))
    w1b = f32z((128, 128))
    sd = f32z((2, 128))
    w2a = f32z((128, 128))
    w2b = f32z((128, 128))
    for t in range(2):
        o = 64 * t
        ct = ct.at[:, o + 32:o + 48].set(eye16)
        ssel = ssel.at[o + 32:o + 64, t].set(1.0)
        w1b = w1b.at[o + 32:o + 48, o:o + 64].set(w1_pad)
        sd = sd.at[t, o:o + 64].set(w1d)
        w2a = w2a.at[o:o + 64, o:o + 32].set(w2_bot[:, 0:32])
        w2b = w2b.at[o:o + 64, o:o + 32].set(w2_bot[:, 32:64])
    b1q = jnp.concatenate([b1r, b1r], axis=1)

    phase_nodes = [4000, 4000, 2000]
    idx_all = knn_idx.reshape(-1)
    phases = []
    noff = 0
    for n_p in phase_nodes:
        e_p = n_p * k_
        cpt_p = (e_p + _NW * _CH - 1) // (_NW * _CH)
        ep_p = cpt_p * _NW * _CH
        idx3d = jnp.pad(
            idx_all[noff * k_:noff * k_ + e_p], (0, ep_p - e_p)
        ).reshape(_NW, cpt_p, _CH)
        phases.append((noff, n_p, cpt_p, ep_p, idx3d))
        noff += n_p

    pts_pad = f32z((n_, 32)).at[:, :3].set(pts)
    pts16 = pts_pad[:, :16]
    tbl = pl.pallas_call(
        _tbl_body,
        out_shape=jax.ShapeDtypeStruct((n_, 2 * dh // 2), jnp.float32),
    )(feats, w2_top, b2r, pts_pad)

    galls = [
        _make_gather(ep_p, cpt_p)(idx3d, tbl).reshape(ep_p // 2, 128)
        for (_, _, cpt_p, ep_p, idx3d) in phases
    ]

    blk = 200
    e2b = blk * k_ // 2
    full = lambda i: (0, 0)

    def finish_call(noff, n_p):
        off = noff // blk
        nb = n_p // blk
        return pl.pallas_call(
            _make_finish(blk, k_),
            grid=(nb,),
            in_specs=[
                pl.BlockSpec((e2b, 128), lambda i: (i, 0)),
                pl.BlockSpec((blk, 16), lambda i, o=off: (i + o, 0)),
                pl.BlockSpec((16, 128), full),
                pl.BlockSpec((1, 128), full),
                pl.BlockSpec((1, 128), full),
                pl.BlockSpec((128, 2), full),
                pl.BlockSpec((128, 128), full),
                pl.BlockSpec((2, 128), full),
                pl.BlockSpec((1, 128), full),
                pl.BlockSpec((128, 128), full),
                pl.BlockSpec((128, 128), full),
            ],
            out_specs=pl.BlockSpec((blk, dh), lambda i: (i, 0)),
            out_shape=jax.ShapeDtypeStruct((n_p, dh), jnp.float32),
        )

    outs = [
        finish_call(noff, n_p)(galls[p], pts16, ct,
                               mpt, mpr, ssel, w1b, sd, b1q, w2a, w2b)
        for p, (noff, n_p, _, _, _) in enumerate(phases)
    ]

    return jnp.concatenate(outs, axis=0).reshape(b_, n_, dh)
